# baseline (speedup 1.0000x reference)
import functools

import jax
import jax.numpy as jnp
from jax import lax
from jax.experimental import pallas as pl
from jax.experimental.pallas import tpu as pltpu
from jax.experimental.pallas import tpu_sc as plsc

B = 4096
DIM_CIDX = 16
NUM_SLOT = 1000
DIM_SLOT = 7
L = 16

OFF_W1 = 0
OFF_B1 = 576
OFF_W2 = 600
OFF_B2 = 888
OFF_W3 = 900
OFF_B3 = 912
WLEN = 928


def _sc_geometry():
    try:
        info = plsc.get_sparse_core_info()
        return info.num_cores, info.num_subcores
    except Exception:
        return 2, 16


def _splat(v):
    return jnp.full((L,), v, jnp.int32)


def _make_kernel(nc, ns):
    nw = nc * ns
    bpw = B // nw
    ngroups = bpw // L
    mesh = plsc.VectorSubcoreMesh(
        core_axis_name="c", subcore_axis_name="s",
        num_cores=nc, num_subcores=ns)

    @functools.partial(
        pl.kernel,
        out_type=jax.ShapeDtypeStruct((B,), jnp.float32),
        mesh=mesh,
        compiler_params=pltpu.CompilerParams(
            use_tc_tiling_on_sc=False, needs_layout_passes=False),
        scratch_types=[
            pltpu.VMEM((bpw, 3), jnp.int32),
            pltpu.VMEM((bpw,), jnp.int32),
            pltpu.VMEM((NUM_SLOT, DIM_SLOT), jnp.float32),
            pltpu.VMEM((WLEN * L,), jnp.float32),
            pltpu.VMEM((bpw, DIM_CIDX), jnp.float32),
            pltpu.VMEM((bpw,), jnp.float32),
            pltpu.SemaphoreType.DMA,
            pltpu.SemaphoreType.DMA,
            pltpu.SemaphoreType.DMA,
        ],
    )
    def sc_kernel(x_hbm, ctab_hbm, stab_hbm, w_hbm, out_hbm,
                  xv, ci_v, stab_v, wf_v, crows, yv, sem_w, sem_s, sem_c):
        wid = lax.axis_index("s") * nc + lax.axis_index("c")
        base = wid * bpw
        pltpu.sync_copy(x_hbm.at[pl.ds(base, bpw), :], xv)
        cp_w = pltpu.async_copy(w_hbm, wf_v, sem_w)
        cp_s = pltpu.async_copy(stab_hbm, stab_v, sem_s)

        iota = lax.iota(jnp.int32, L)

        def build_cidx(g, carry):
            lrows = iota + g * L
            c16 = plsc.load_gather(xv, [lrows, _splat(0)])
            ci_v[pl.ds(g * L, L)] = c16
            return carry

        lax.fori_loop(0, ngroups, build_cidx, 0, unroll=True)
        cp_c = pltpu.async_copy(ctab_hbm.at[ci_v], crows, sem_c)
        cp_s.wait()
        cp_w.wait()
        cp_c.wait()

        def wv(o):
            return wf_v[pl.ds(o * L, L)]

        def group_body(g, carry):
            lrows = iota + g * L
            slot16 = plsc.load_gather(xv, [lrows, _splat(1)])
            f16 = plsc.load_gather(xv, [lrows, _splat(2)]).astype(jnp.float32)
            t = [plsc.load_gather(crows, [lrows, _splat(k)])
                 for k in range(DIM_CIDX)]
            t += [plsc.load_gather(stab_v, [slot16, _splat(j)])
                  for j in range(DIM_SLOT)]
            t.append(f16)

            h1 = []
            for j in range(24):
                acc = t[0] * wv(OFF_W1 + j * 24)
                for k in range(1, 24):
                    acc = acc + t[k] * wv(OFF_W1 + j * 24 + k)
                h1.append(jnp.maximum(acc + wv(OFF_B1 + j), 0.0))

            h2 = []
            for j in range(12):
                acc = h1[0] * wv(OFF_W2 + j * 24)
                for k in range(1, 24):
                    acc = acc + h1[k] * wv(OFF_W2 + j * 24 + k)
                h2.append(jnp.maximum(acc + wv(OFF_B2 + j), 0.0))

            acc = h2[0] * wv(OFF_W3)
            for k in range(1, 12):
                acc = acc + h2[k] * wv(OFF_W3 + k)
            yv[pl.ds(g * L, L)] = acc + wv(OFF_B3)
            return carry

        lax.fori_loop(0, ngroups, group_body, 0)
        pltpu.sync_copy(yv, out_hbm.at[pl.ds(base, bpw)])

    return sc_kernel


def kernel(x, emb_cidx, emb_slot, W1, b1, W2, b2, W3, b3):
    wflat = jnp.concatenate([
        W1.reshape(-1), b1, W2.reshape(-1), b2, W3.reshape(-1), b3,
        jnp.zeros((WLEN - OFF_B3 - 1,), jnp.float32)])
    wsplat = jnp.repeat(wflat, L)
    nc, ns = _sc_geometry()
    return _make_kernel(nc, ns)(x, emb_cidx, emb_slot, wsplat)


# speedup vs baseline: 1.1266x; 1.1266x over previous
import functools

import jax
import jax.numpy as jnp
from jax import lax
from jax.experimental import pallas as pl
from jax.experimental.pallas import tpu as pltpu
from jax.experimental.pallas import tpu_sc as plsc

B = 4096
DIM_CIDX = 16
NUM_SLOT = 1000
DIM_SLOT = 7
DIN = 24
DH1 = 24
DH2 = 12
L = 16

OFF_W1 = 16
OFF_B1 = 16 + 768
OFF_W2 = 816
OFF_B2 = 816 + 384
OFF_W3 = 1216
OFF_B3 = 1232
WLEN = 1248


def _sc_geometry():
    try:
        info = plsc.get_sparse_core_info()
        return info.num_cores, info.num_subcores
    except Exception:
        return 2, 16


def _splat(v):
    return jnp.full((L,), v, jnp.int32)


def _make_kernel(nc, ns):
    nw = nc * ns
    bpw = B // nw
    ng = bpw // L
    mesh = plsc.VectorSubcoreMesh(
        core_axis_name="c", subcore_axis_name="s",
        num_cores=nc, num_subcores=ns)

    @functools.partial(
        pl.kernel,
        out_type=jax.ShapeDtypeStruct((B,), jnp.float32),
        mesh=mesh,
        compiler_params=pltpu.CompilerParams(
            use_tc_tiling_on_sc=False, needs_layout_passes=False),
        scratch_types=[
            pltpu.VMEM((bpw, 3), jnp.int32),
            pltpu.VMEM((bpw,), jnp.int32),
            pltpu.VMEM((NUM_SLOT, DIM_SLOT), jnp.float32),
            pltpu.VMEM((WLEN,), jnp.float32),
            pltpu.VMEM((bpw, DIM_CIDX), jnp.float32),
            pltpu.VMEM((DIN, bpw), jnp.float32),
            pltpu.VMEM((DH1, bpw), jnp.float32),
            pltpu.VMEM((DH2, bpw), jnp.float32),
            pltpu.VMEM((bpw,), jnp.float32),
            pltpu.SemaphoreType.DMA,
            pltpu.SemaphoreType.DMA,
            pltpu.SemaphoreType.DMA,
        ],
    )
    def sc_kernel(x_hbm, cidx_hbm, ctab_hbm, stab_hbm, w_hbm, out_hbm,
                  xv, ci_v, stab_v, wf_v, crows, t_ref, h1_ref, h2_ref, yv,
                  sem_w, sem_s, sem_c):
        wid = lax.axis_index("s") * nc + lax.axis_index("c")
        base = wid * bpw
        pltpu.sync_copy(x_hbm.at[pl.ds(base, bpw), :], xv)
        cp_w = pltpu.async_copy(w_hbm, wf_v, sem_w)
        cp_s = pltpu.async_copy(stab_hbm, stab_v, sem_s)

        iota = lax.iota(jnp.int32, L)

        pltpu.sync_copy(cidx_hbm.at[pl.ds(base, bpw)], ci_v)
        cp_c = pltpu.async_copy(ctab_hbm.at[ci_v], crows, sem_c)
        cp_s.wait()
        cp_w.wait()
        cp_c.wait()

        for g in range(ng):
            lrows = iota + g * L
            sl = pl.ds(g * L, L)
            for k in range(DIM_CIDX):
                t_ref[k, sl] = plsc.load_gather(crows, [lrows, _splat(k)])
            slot16 = plsc.load_gather(xv, [lrows, _splat(1)])
            for j in range(DIM_SLOT):
                t_ref[DIM_CIDX + j, sl] = plsc.load_gather(
                    stab_v, [slot16, _splat(j)])
            t_ref[DIN - 1, sl] = plsc.load_gather(
                xv, [lrows, _splat(2)]).astype(jnp.float32)

        ck = [_splat(k) for k in range(L)]
        gdn = lax.GatherDimensionNumbers(
            offset_dims=(), collapsed_slice_dims=(0,), start_index_map=(0,))

        def lane(vec, k):
            return lax.gather(
                vec, ck[k][:, None], gdn, (1,),
                mode=lax.GatherScatterMode.PROMISE_IN_BOUNDS)

        def dense_block(src_ref, kdim, off_w, off_b, dst_ref, jblk, j0):
            wlo = [wf_v[pl.ds(off_w + (j0 + jj) * 32, L)]
                   for jj in range(jblk)]
            whi = [wf_v[pl.ds(off_w + (j0 + jj) * 32 + L, L)]
                   for jj in range(jblk)]
            tk = [src_ref[0, pl.ds(m * L, L)] for m in range(ng)]
            accs = []
            for jj in range(jblk):
                w = lane(wlo[jj], 0)
                accs.append([tk[m] * w for m in range(ng)])
            for k in range(1, kdim):
                tk = [src_ref[k, pl.ds(m * L, L)] for m in range(ng)]
                for jj in range(jblk):
                    w = (lane(wlo[jj], k) if k < L
                         else lane(whi[jj], k - L))
                    accs[jj] = [accs[jj][m] + tk[m] * w
                                for m in range(ng)]
            blo = wf_v[pl.ds(off_b, L)]
            bhi = wf_v[pl.ds(off_b + L, L)]
            for jj in range(jblk):
                row = j0 + jj
                bw = lane(blo, row) if row < L else lane(bhi, row - L)
                for m in range(ng):
                    v = jnp.maximum(accs[jj][m] + bw, 0.0)
                    dst_ref[row, pl.ds(m * L, L)] = v

        for b in range(DH1 // 4):
            dense_block(t_ref, DIN, OFF_W1, OFF_B1, h1_ref, 4, b * 4)
        for b in range(DH2 // 4):
            dense_block(h1_ref, DH1, OFF_W2, OFF_B2, h2_ref, 4, b * 4)

        w3v = wf_v[pl.ds(OFF_W3, L)]
        b3v = wf_v[pl.ds(OFF_B3, L)]
        w = lane(w3v, 0)
        acc = [h2_ref[0, pl.ds(m * L, L)] * w for m in range(ng)]
        for k in range(1, DH2):
            w = lane(w3v, k)
            acc = [acc[m] + h2_ref[k, pl.ds(m * L, L)] * w for m in range(ng)]
        b3 = lane(b3v, 0)
        for m in range(ng):
            yv[pl.ds(m * L, L)] = acc[m] + b3

        pltpu.sync_copy(yv, out_hbm.at[pl.ds(base, bpw)])

    return sc_kernel


def kernel(x, emb_cidx, emb_slot, W1, b1, W2, b2, W3, b3):
    z = jnp.zeros
    wflat = jnp.concatenate([
        z((16,), jnp.float32),
        jnp.pad(W1, ((0, 0), (0, 8))).reshape(-1),
        b1, z((8,), jnp.float32),
        jnp.pad(W2, ((0, 0), (0, 8))).reshape(-1),
        b2, z((4,), jnp.float32),
        W3.reshape(-1), z((4,), jnp.float32),
        b3, z((15,), jnp.float32)])
    nc, ns = _sc_geometry()
    return _make_kernel(nc, ns)(x, x[:, 0], emb_cidx, emb_slot, wflat)
